# baseline (speedup 1.0000x reference)
import functools

import jax
import jax.numpy as jnp
from jax import lax
from jax.experimental import pallas as pl
from jax.experimental.pallas import tpu as pltpu
from jax.experimental.pallas import tpu_sc as plsc

N_CATEG = 26
NUM_CATS = 100
DIM = 128
BATCH = 16384
NON_CATEG = DIM - N_CATEG
OUT_D = NON_CATEG + N_CATEG * NUM_CATS
P_W = 32

L = 16
NC = 2
NS = 16
NW = NC * NS
ROWS_PER_W = BATCH // NW

RB = 512


def _sc_positions_body(x_hbm, p_hbm, xv, pv):
    wid = lax.axis_index("s") * NC + lax.axis_index("c")
    iota = lax.iota(jnp.int32, L)
    mask_hi = iota < (N_CATEG - L)
    base = wid * ROWS_PER_W
    pltpu.sync_copy(x_hbm.at[pl.ds(base, ROWS_PER_W)], xv)

    def _row(r, _):
        rr = iota * 0 + r
        cat_lo = plsc.load_gather(xv, [rr, iota])
        cat_hi = plsc.load_gather(xv, [rr, iota + L])
        pos_lo = NON_CATEG + iota * NUM_CATS + cat_lo.astype(jnp.int32)
        pos_hi = NON_CATEG + (iota + L) * NUM_CATS + cat_hi.astype(jnp.int32)
        pos_hi = jnp.where(mask_hi, pos_hi, -1)
        plsc.store_scatter(pv, [rr, iota], pos_lo)
        plsc.store_scatter(pv, [rr, iota + L], pos_hi)
        return 0
    lax.fori_loop(0, ROWS_PER_W, _row, 0)

    pltpu.sync_copy(pv, p_hbm.at[pl.ds(base, ROWS_PER_W)])


def _tc_materialize_body(xt_ref, pt_ref, o_ref):
    o_ref[:NON_CATEG, :] = xt_ref[N_CATEG:, :]
    row2 = lax.broadcasted_iota(jnp.int32, (NUM_CATS, RB), 0)
    for i in range(N_CATEG):
        seg = (row2 + (NON_CATEG + i * NUM_CATS) == pt_ref[i:i + 1, :])
        o_ref[NON_CATEG + i * NUM_CATS:
              NON_CATEG + (i + 1) * NUM_CATS, :] = seg.astype(jnp.float32)


def kernel(X, emb_tables):
    del emb_tables
    mesh = plsc.VectorSubcoreMesh(core_axis_name="c", subcore_axis_name="s")
    sc_positions = functools.partial(
        pl.kernel,
        out_type=jax.ShapeDtypeStruct((BATCH, P_W), jnp.int32),
        mesh=mesh,
        compiler_params=pltpu.CompilerParams(needs_layout_passes=False),
        scratch_types=[
            pltpu.VMEM((ROWS_PER_W, DIM), jnp.float32),
            pltpu.VMEM((ROWS_PER_W, P_W), jnp.int32),
        ],
    )(_sc_positions_body)
    P = sc_positions(X)

    OT = pl.pallas_call(
        _tc_materialize_body,
        grid=(BATCH // RB,),
        in_specs=[pl.BlockSpec((DIM, RB), lambda i: (0, i)),
                  pl.BlockSpec((P_W, RB), lambda i: (0, i))],
        out_specs=pl.BlockSpec((OUT_D, RB), lambda i: (0, i)),
        out_shape=jax.ShapeDtypeStruct((OUT_D, BATCH), jnp.float32),
    )(X.T, P.T)
    return OT.T


# speedup vs baseline: 1.0216x; 1.0216x over previous
import functools

import jax
import jax.numpy as jnp
from jax import lax
from jax.experimental import pallas as pl
from jax.experimental.pallas import tpu as pltpu
from jax.experimental.pallas import tpu_sc as plsc

N_CATEG = 26
NUM_CATS = 100
DIM = 128
BATCH = 16384
NON_CATEG = DIM - N_CATEG
OUT_D = NON_CATEG + N_CATEG * NUM_CATS
P_W = 32

L = 16
NC = 2
NS = 16
NW = NC * NS
ROWS_PER_W = BATCH // NW

RB = 1024


def _sc_positions_body(x_hbm, p_hbm, xv, pv):
    wid = lax.axis_index("s") * NC + lax.axis_index("c")
    iota = lax.iota(jnp.int32, L)
    mask_hi = iota < (N_CATEG - L)
    base = wid * ROWS_PER_W
    pltpu.sync_copy(x_hbm.at[pl.ds(base, ROWS_PER_W)], xv)

    def _row(r, _):
        rr = iota * 0 + r
        cat_lo = plsc.load_gather(xv, [rr, iota])
        cat_hi = plsc.load_gather(xv, [rr, iota + L])
        pos_lo = NON_CATEG + iota * NUM_CATS + cat_lo.astype(jnp.int32)
        pos_hi = NON_CATEG + (iota + L) * NUM_CATS + cat_hi.astype(jnp.int32)
        pos_hi = jnp.where(mask_hi, pos_hi, -1)
        plsc.store_scatter(pv, [rr, iota], pos_lo)
        plsc.store_scatter(pv, [rr, iota + L], pos_hi)
        return 0
    lax.fori_loop(0, ROWS_PER_W, _row, 0)

    pltpu.sync_copy(pv, p_hbm.at[pl.ds(base, ROWS_PER_W)])


def _tc_materialize_body(xt_ref, pt_ref, o_ref):
    o_ref[:NON_CATEG, :] = xt_ref[N_CATEG:, :]
    row2 = lax.broadcasted_iota(jnp.int32, (NUM_CATS, RB), 0)
    for i in range(N_CATEG):
        seg = (row2 + (NON_CATEG + i * NUM_CATS) == pt_ref[i:i + 1, :])
        o_ref[NON_CATEG + i * NUM_CATS:
              NON_CATEG + (i + 1) * NUM_CATS, :] = seg.astype(jnp.float32)


def kernel(X, emb_tables):
    del emb_tables
    mesh = plsc.VectorSubcoreMesh(core_axis_name="c", subcore_axis_name="s")
    sc_positions = functools.partial(
        pl.kernel,
        out_type=jax.ShapeDtypeStruct((BATCH, P_W), jnp.int32),
        mesh=mesh,
        compiler_params=pltpu.CompilerParams(needs_layout_passes=False),
        scratch_types=[
            pltpu.VMEM((ROWS_PER_W, DIM), jnp.float32),
            pltpu.VMEM((ROWS_PER_W, P_W), jnp.int32),
        ],
    )(_sc_positions_body)
    P = sc_positions(X)

    OT = pl.pallas_call(
        _tc_materialize_body,
        grid=(BATCH // RB,),
        in_specs=[pl.BlockSpec((DIM, RB), lambda i: (0, i)),
                  pl.BlockSpec((P_W, RB), lambda i: (0, i))],
        out_specs=pl.BlockSpec((OUT_D, RB), lambda i: (0, i)),
        out_shape=jax.ShapeDtypeStruct((OUT_D, BATCH), jnp.float32),
    )(X.T, P.T)
    return OT.T
